# Initial kernel scaffold; baseline (speedup 1.0000x reference)
import jax
import jax.numpy as jnp
from jax.experimental import pallas as pl


def kernel(x, edge_index_a, edge_index_b, batch, W_in, b_in, W1, b1, g1, be1, W2, b2, gno, bno, Wc1, bc1, gc, bec, Wc2, bc2):
    raise NotImplementedError("write your pallas kernel here")


# speedup vs baseline: 2.4999x; 2.4999x over previous
import functools

import jax
import jax.numpy as jnp
from jax import lax
from jax.experimental import pallas as pl
from jax.experimental.pallas import tpu as pltpu
from jax.experimental.pallas import tpu_sc as plsc

_N, _E, _D, _H, _G, _C, _L, _T = 10000, 320000, 128, 128, 64, 2, 3, 2
_NT = 16
_K = 128
_EPT = _E // _NT
_CHB = 16
_NBLK = -(-_EPT // (_K * _CHB))
_NCH = _NBLK * _CHB
_PAD = _NCH * _K - _EPT
_ACC_ROWS = 10240
_TRASH = 10200
_RPT = _ACC_ROWS // _NT
_DCH = 64
_EPS = 1e-5


def _prep_edges(ei_a, ei_b):
    src = jnp.stack([ei_a[0], ei_b[0]]).astype(jnp.int32).reshape(_T, _NT, _EPT)
    dst = jnp.stack([ei_a[1], ei_b[1]]).astype(jnp.int32).reshape(_T, _NT, _EPT)
    src = jnp.concatenate(
        [src, jnp.zeros((_T, _NT, _PAD), jnp.int32)], axis=-1)
    dst = jnp.concatenate(
        [dst, jnp.full((_T, _NT, _PAD), _TRASH, jnp.int32)], axis=-1)
    return (src.reshape(_T, _NT, _NCH, _K), dst.reshape(_T, _NT, _NCH, _K))


def _seg_sums(h, src, dst):
    mesh = plsc.VectorSubcoreMesh(core_axis_name="c", subcore_axis_name="s")

    @functools.partial(
        pl.kernel,
        out_type=jax.ShapeDtypeStruct((_T, _ACC_ROWS, _H), jnp.float32),
        mesh=mesh,
        scratch_types=[
            pltpu.VMEM((_CHB, _K), jnp.int32),
            pltpu.VMEM((_CHB, _K), jnp.int32),
            pltpu.VMEM((_K, _H), jnp.float32),
            pltpu.VMEM((_DCH, _H), jnp.float32),
            pltpu.VMEM_SHARED((_ACC_ROWS, _H), jnp.float32),
            pltpu.SemaphoreType.DMA,
        ],
    )
    def k(h_hbm, src_hbm, dst_hbm, out_hbm, sidx, didx, rows, stage, acc, sem):
        c = lax.axis_index("c")
        s = lax.axis_index("s")

        def zrow(i, carry):
            for j in range(_H // 16):
                stage[i, pl.ds(j * 16, 16)] = jnp.zeros((16,), jnp.float32)
            return carry
        lax.fori_loop(0, _DCH, zrow, 0)

        def zchunk(r, carry):
            pltpu.sync_copy(stage, acc.at[pl.ds(s * _RPT + r * _DCH, _DCH)])
            return carry
        lax.fori_loop(0, _RPT // _DCH, zchunk, 0)

        plsc.subcore_barrier()

        def block(b, carry):
            pltpu.sync_copy(src_hbm.at[c, s, pl.ds(b * _CHB, _CHB)], sidx)
            pltpu.sync_copy(dst_hbm.at[c, s, pl.ds(b * _CHB, _CHB)], didx)

            def chunk(i, carry2):
                pltpu.async_copy(h_hbm.at[sidx.at[i]], rows, sem).wait()
                pltpu.sync_copy(rows, acc.at[didx.at[i]], add=True)
                return carry2
            return lax.fori_loop(0, _CHB, chunk, carry)
        lax.fori_loop(0, _NBLK, block, 0)
        plsc.subcore_barrier()

        def dump(r, carry):
            base = s * _RPT + r * _DCH
            pltpu.sync_copy(acc.at[pl.ds(base, _DCH)], stage)
            pltpu.sync_copy(stage, out_hbm.at[c, pl.ds(base, _DCH)])
            return carry
        lax.fori_loop(0, _RPT // _DCH, dump, 0)

    return k(h, src, dst)


def _bn(u, g, b):
    mu = jnp.mean(u, axis=0, keepdims=True)
    d = u - mu
    var = jnp.mean(d * d, axis=0, keepdims=True)
    return d * lax.rsqrt(var + _EPS) * g + b


def _dot(a, b):
    return jnp.dot(a, b, preferred_element_type=jnp.float32)


def _dot_hi(a, b):
    return jnp.dot(a, b, preferred_element_type=jnp.float32,
                   precision=lax.Precision.HIGHEST)


def _gin(h, m, w1, b1, g1v, be1v, w2, b2):
    u = _dot(h + m, w1) + b1
    u = jnp.maximum(_bn(u, g1v, be1v), 0.0)
    return _dot(u, w2) + b2


def _in_proj(x, w, b):
    def body(x_ref, w_ref, b_ref, o_ref):
        o_ref[...] = _dot(x_ref[...], w_ref[...]) + b_ref[...]
    return pl.pallas_call(
        body, out_shape=jax.ShapeDtypeStruct((_N, _H), jnp.float32))(x, w, b)


def _layer(h, m, pa, pb, gno_l, bno_l):
    def body(h_ref, m_ref, w1a, b1a, g1a, be1a, w2a, b2a,
             w1b, b1b, g1b, be1b, w2b, b2b, gno_r, bno_r, o_ref):
        hh = h_ref[...]
        oa = _gin(hh, m_ref[0, :_N], w1a[...], b1a[...], g1a[...], be1a[...],
                  w2a[...], b2a[...])
        ob = _gin(hh, m_ref[1, :_N], w1b[...], b1b[...], g1b[...], be1b[...],
                  w2b[...], b2b[...])
        o_ref[...] = jnp.maximum(_bn(oa + ob, gno_r[...], bno_r[...]), 0.0)
    return pl.pallas_call(
        body, out_shape=jax.ShapeDtypeStruct((_N, _H), jnp.float32))(
        h, m, *pa, *pb, gno_l, bno_l)


def _layer_final(h, m, pa, pb, gno_l, bno_l, batch2d,
                 wc1, bc1, gcv, becv, wc2, bc2):
    def body(h_ref, m_ref, w1a, b1a, g1a, be1a, w2a, b2a,
             w1b, b1b, g1b, be1b, w2b, b2b, gno_r, bno_r,
             batch_ref, wc1_r, bc1_r, gc_r, bec_r, wc2_r, bc2_r, o_ref):
        hh = h_ref[...]
        oa = _gin(hh, m_ref[0, :_N], w1a[...], b1a[...], g1a[...], be1a[...],
                  w2a[...], b2a[...])
        ob = _gin(hh, m_ref[1, :_N], w1b[...], b1b[...], g1b[...], be1b[...],
                  w2b[...], b2b[...])
        hn = jnp.maximum(_bn(oa + ob, gno_r[...], bno_r[...]), 0.0)
        onehot = jnp.where(
            lax.broadcasted_iota(jnp.int32, (_G, _N), 0) == batch_ref[...],
            1.0, 0.0)
        counts = jnp.maximum(jnp.sum(onehot, axis=1, keepdims=True), 1.0)
        pooled = _dot_hi(onehot, hn) / counts
        zc = _dot(pooled, wc1_r[...]) + bc1_r[...]
        zc = jnp.maximum(_bn(zc, gc_r[...], bec_r[...]), 0.0)
        o_ref[...] = _dot(zc, wc2_r[...]) + bc2_r[...]
    return pl.pallas_call(
        body, out_shape=jax.ShapeDtypeStruct((_G, _C), jnp.float32))(
        h, m, *pa, *pb, gno_l, bno_l, batch2d, wc1, bc1, gcv, becv, wc2, bc2)


def kernel(x, edge_index_a, edge_index_b, batch, W_in, b_in, W1, b1, g1, be1,
           W2, b2, gno, bno, Wc1, bc1, gc, bec, Wc2, bc2):
    src, dst = _prep_edges(edge_index_a, edge_index_b)
    batch2d = batch.astype(jnp.int32).reshape(1, _N)
    r1 = lambda v: v.reshape(1, -1)

    h = _in_proj(x, W_in, r1(b_in))
    out = None
    for l in range(_L):
        m = _seg_sums(h, src, dst)
        pa = (W1[l, 0], r1(b1[l, 0]), r1(g1[l, 0]), r1(be1[l, 0]),
              W2[l, 0], r1(b2[l, 0]))
        pb = (W1[l, 1], r1(b1[l, 1]), r1(g1[l, 1]), r1(be1[l, 1]),
              W2[l, 1], r1(b2[l, 1]))
        if l < _L - 1:
            h = _layer(h, m, pa, pb, r1(gno[l]), r1(bno[l]))
        else:
            out = _layer_final(h, m, pa, pb, r1(gno[l]), r1(bno[l]), batch2d,
                               Wc1, r1(bc1), r1(gc), r1(bec), Wc2, r1(bc2))
    return out
